# Initial kernel scaffold; baseline (speedup 1.0000x reference)
import jax
import jax.numpy as jnp
from jax.experimental import pallas as pl


def kernel(c, p, n, center_w, context_w):
    raise NotImplementedError("write your pallas kernel here")


# speedup vs baseline: 4.5435x; 4.5435x over previous
import functools

import jax
import jax.numpy as jnp
from jax import lax
from jax.experimental import pallas as pl
from jax.experimental.pallas import tpu as pltpu
from jax.experimental.pallas import tpu_sc as plsc

VOCAB = 100000
DIM = 128
B = 16384
K = 20
J = K + 1
OUTW = 32
LANES = 16
NSEG = DIM // LANES

NC = 2
NS = 16
NW = NC * NS

GCH = 112


def _sc_body(cb, rpw, c_hbm, np_hbm, cen_hbm, ctx_hbm, out_hbm,
             cidx_v, npidx_v, ce_v, ctx_v, tr_v, out_v, sem):
    nchunk = rpw // cb
    ng = (cb * J) // GCH
    wid = lax.axis_index("s") * NC + lax.axis_index("c")
    iota = lax.iota(jnp.int32, LANES)
    zero = jnp.zeros((LANES,), jnp.float32)
    for r in range(J, 2 * LANES):
        tr_v[pl.ds(r * LANES, LANES)] = zero

    def chunk_body(t, carry):
        base = wid * rpw + t * cb
        pltpu.sync_copy(c_hbm.at[pl.ds(base, cb)], cidx_v)
        pltpu.sync_copy(np_hbm.at[pl.ds(base * J, cb * J)], npidx_v)
        cp_ce = pltpu.async_copy(cen_hbm.at[cidx_v], ce_v, sem)
        cps = [
            pltpu.async_copy(
                ctx_hbm.at[npidx_v.at[pl.ds(g * GCH, GCH)]],
                ctx_v.at[pl.ds(g * GCH, GCH)],
                sem,
            )
            for g in range(ng)
        ]
        cp_ce.wait()
        for cp in cps:
            cp.wait()

        def row_body(b, c2):
            ce = [ce_v[b, pl.ds(LANES * i, LANES)] for i in range(NSEG)]
            for j in range(J):
                r = b * J + j
                acc = ce[0] * ctx_v[r, pl.ds(0, LANES)]
                for i in range(1, NSEG):
                    acc = acc + ce[i] * ctx_v[r, pl.ds(LANES * i, LANES)]
                tr_v[pl.ds(j * LANES, LANES)] = acc
            rowbase = iota * LANES
            for h in range(2):
                s = plsc.load_gather(tr_v, [rowbase + (h * LANES * LANES)])
                for m in range(1, LANES):
                    s = s + plsc.load_gather(
                        tr_v, [rowbase + (h * LANES * LANES + m)]
                    )
                out_v[b, pl.ds(LANES * h, LANES)] = s
            return c2

        lax.fori_loop(0, cb, row_body, 0, unroll=False)
        pltpu.sync_copy(out_v, out_hbm.at[pl.ds(base, cb)])
        return carry

    lax.fori_loop(0, nchunk, chunk_body, 0, unroll=False)


def _make_sc_dots(b_total, cb, interpret=False):
    rpw = b_total // NW
    return functools.partial(
        pl.kernel,
        out_type=jax.ShapeDtypeStruct((b_total, OUTW), jnp.float32),
        mesh=plsc.VectorSubcoreMesh(
            core_axis_name="c", subcore_axis_name="s",
            num_cores=NC, num_subcores=NS,
        ),
        scratch_types=[
            pltpu.VMEM((cb,), jnp.int32),
            pltpu.VMEM((cb * J,), jnp.int32),
            pltpu.VMEM((cb, DIM), jnp.float32),
            pltpu.VMEM((cb * J, DIM), jnp.float32),
            pltpu.VMEM((2 * LANES * LANES,), jnp.float32),
            pltpu.VMEM((cb, OUTW), jnp.float32),
            pltpu.SemaphoreType.DMA,
        ],
        compiler_params=pltpu.CompilerParams(needs_layout_passes=False),
        interpret=interpret,
    )(functools.partial(_sc_body, cb, rpw))


def _loss_body(bk, x_ref, o_ref):
    x = x_ref[...]
    col = lax.broadcasted_iota(jnp.int32, x.shape, 1)
    sign = jnp.where(col == K, 1.0, -1.0).astype(jnp.float32)
    w = jnp.where(
        col == K, 1.0 / bk, jnp.where(col < K, 1.0 / (bk * K), 0.0)
    ).astype(jnp.float32)
    t = sign * x
    ls = jnp.minimum(t, 0.0) - jnp.log1p(jnp.exp(-jnp.abs(t)))
    o_ref[0, 0] = -jnp.sum(w * ls)


def _loss_from_dots(dots, interpret=False):
    bk = dots.shape[0]
    out = pl.pallas_call(
        functools.partial(_loss_body, bk),
        out_shape=jax.ShapeDtypeStruct((1, 1), jnp.float32),
        out_specs=pl.BlockSpec(memory_space=pltpu.SMEM),
        interpret=interpret,
    )(dots)
    return out[0, 0]


@jax.jit
def kernel(c, p, n, center_w, context_w):
    c = c.astype(jnp.int32)
    np_idx = jnp.concatenate(
        [n.astype(jnp.int32), p.astype(jnp.int32)[:, None]], axis=1
    ).reshape(-1)
    dots = _make_sc_dots(B, 32)(c, np_idx, center_w, context_w)
    return _loss_from_dots(dots)
